# Initial kernel scaffold; baseline (speedup 1.0000x reference)
import jax
import jax.numpy as jnp
from jax.experimental import pallas as pl


def kernel(depot_xy, customer_xy, demand, params):
    raise NotImplementedError("write your pallas kernel here")


# speedup vs baseline: 84.6555x; 84.6555x over previous
import numpy as np

import jax
import jax.numpy as jnp
from jax.experimental import pallas as pl

B = 100
N = 101
E = 128
HID = 512
NODES = B * N
GB = 10
STEPS = B // GB
EPS = 1e-5
F32 = jnp.float32


def _tri_matrix():
    j = np.arange(N, dtype=np.float64)
    dinv = 1.0 / np.sqrt(j + 1.0)
    a = np.tril(np.ones((N, N))) * (dinv[:, None] * dinv[None, :])
    return jnp.asarray(a, dtype=F32)


def _bn_coefs(s_ref, q_ref, g_ref, b_ref):
    mu = s_ref[...] * (1.0 / NODES)
    var = q_ref[...] * (1.0 / NODES) - mu * mu
    scale = g_ref[...] * jax.lax.rsqrt(var + EPS)
    shift = b_ref[...] - mu * scale
    return scale, shift


def _acc_stats(step, sa, qa, so_ref, qo_ref):
    @pl.when(step == 0)
    def _():
        so_ref[...] = sa
        qo_ref[...] = qa

    @pl.when(step != 0)
    def _():
        so_ref[...] += sa
        qo_ref[...] += qa


def _embed_body(depot_ref, cust_ref, wd_ref, bd_ref, wi_ref, bi_ref,
                d_ref, c_ref):
    d_ref[...] = jnp.dot(depot_ref[...], wd_ref[...],
                         preferred_element_type=F32) + bd_ref[...]
    c_ref[...] = jnp.dot(cust_ref[...], wi_ref[...],
                         preferred_element_type=F32) + bi_ref[...]


def _gcn0_body(x_ref, wg_ref, bg_ref, a_ref, y_ref, so_ref, qo_ref):
    step = pl.program_id(0)
    sa = jnp.zeros((1, E), F32)
    qa = jnp.zeros((1, E), F32)
    for i in range(GB):
        z = x_ref[i]
        h = jnp.dot(z, wg_ref[...], preferred_element_type=F32)
        m = jnp.dot(a_ref[...], h, preferred_element_type=F32)
        y = z + m + bg_ref[...]
        y_ref[i] = y
        sa = sa + jnp.sum(y, axis=0, keepdims=True)
        qa = qa + jnp.sum(y * y, axis=0, keepdims=True)
    _acc_stats(step, sa, qa, so_ref, qo_ref)


def _gcn_bn_body(x_ref, s_ref, q_ref, g_ref, b_ref, wg_ref, bg_ref, a_ref,
                 y_ref, so_ref, qo_ref):
    step = pl.program_id(0)
    scale, shift = _bn_coefs(s_ref, q_ref, g_ref, b_ref)
    sa = jnp.zeros((1, E), F32)
    qa = jnp.zeros((1, E), F32)
    for i in range(GB):
        z = x_ref[i] * scale + shift
        h = jnp.dot(z, wg_ref[...], preferred_element_type=F32)
        m = jnp.dot(a_ref[...], h, preferred_element_type=F32)
        y = z + m + bg_ref[...]
        y_ref[i] = y
        sa = sa + jnp.sum(y, axis=0, keepdims=True)
        qa = qa + jnp.sum(y * y, axis=0, keepdims=True)
    _acc_stats(step, sa, qa, so_ref, qo_ref)


def _ff_body(x_ref, s_ref, q_ref, g_ref, b_ref, w1_ref, b1_ref, w2_ref,
             b2_ref, t_ref, so_ref, qo_ref):
    step = pl.program_id(0)
    scale, shift = _bn_coefs(s_ref, q_ref, g_ref, b_ref)
    sa = jnp.zeros((1, E), F32)
    qa = jnp.zeros((1, E), F32)
    for i in range(GB):
        z = x_ref[i] * scale + shift
        h1 = jnp.maximum(
            jnp.dot(z, w1_ref[...], preferred_element_type=F32) + b1_ref[...],
            0.0)
        t = z + jnp.dot(h1, w2_ref[...],
                        preferred_element_type=F32) + b2_ref[...]
        t_ref[i] = t
        sa = sa + jnp.sum(t, axis=0, keepdims=True)
        qa = qa + jnp.sum(t * t, axis=0, keepdims=True)
    _acc_stats(step, sa, qa, so_ref, qo_ref)


def _out_body(x_ref, s_ref, q_ref, g_ref, b_ref, xo_ref, m_ref):
    scale, shift = _bn_coefs(s_ref, q_ref, g_ref, b_ref)
    for i in range(GB):
        z = x_ref[i] * scale + shift
        xo_ref[i] = z
        m_ref[i] = jnp.sum(z, axis=0, keepdims=True) * (1.0 / N)


def _x3_spec():
    return pl.BlockSpec((GB, N, E), lambda i: (i, 0, 0))


def _const_spec(shape):
    nd = len(shape)
    return pl.BlockSpec(shape, lambda i: (0,) * nd)


def kernel(depot_xy, customer_xy, demand, params):
    cust_in = jnp.concatenate([customer_xy, demand[..., None]],
                              axis=-1).reshape(B * (N - 1), 3)
    bd = params["bd"].reshape(1, E)
    bi = params["bi"].reshape(1, E)

    d_emb, c_emb = pl.pallas_call(
        _embed_body,
        grid=(1,),
        in_specs=[_const_spec((B, 2)), _const_spec((B * (N - 1), 3)),
                  _const_spec((2, E)), _const_spec((1, E)),
                  _const_spec((3, E)), _const_spec((1, E))],
        out_specs=[_const_spec((B, E)), _const_spec((B * (N - 1), E))],
        out_shape=[jax.ShapeDtypeStruct((B, E), F32),
                   jax.ShapeDtypeStruct((B * (N - 1), E), F32)],
    )(depot_xy, cust_in, params["Wd"], bd, params["Wi"], bi)

    x3 = jnp.concatenate([d_emb[:, None, :],
                          c_emb.reshape(B, N - 1, E)], axis=1)
    a_mat = _tri_matrix()

    x3_out = jax.ShapeDtypeStruct((B, N, E), F32)
    st_out = jax.ShapeDtypeStruct((1, E), F32)
    stats = None
    prev_gb = None
    for lp in params["layers"]:
        bg = lp["bg"].reshape(1, E)
        if stats is None:
            y3, s1, q1 = pl.pallas_call(
                _gcn0_body,
                grid=(STEPS,),
                in_specs=[_x3_spec(), _const_spec((E, E)),
                          _const_spec((1, E)), _const_spec((N, N))],
                out_specs=[_x3_spec(), _const_spec((1, E)),
                           _const_spec((1, E))],
                out_shape=[x3_out, st_out, st_out],
            )(x3, lp["Wg"], bg, a_mat)
        else:
            s0, q0 = stats
            y3, s1, q1 = pl.pallas_call(
                _gcn_bn_body,
                grid=(STEPS,),
                in_specs=[_x3_spec(), _const_spec((1, E)), _const_spec((1, E)),
                          _const_spec((1, E)), _const_spec((1, E)),
                          _const_spec((E, E)), _const_spec((1, E)),
                          _const_spec((N, N))],
                out_specs=[_x3_spec(), _const_spec((1, E)),
                           _const_spec((1, E))],
                out_shape=[x3_out, st_out, st_out],
            )(x3, s0, q0, prev_gb[0], prev_gb[1], lp["Wg"], bg, a_mat)

        gamma = lp["gamma"].reshape(1, E)
        beta = lp["beta"].reshape(1, E)
        x3, s2, q2 = pl.pallas_call(
            _ff_body,
            grid=(STEPS,),
            in_specs=[_x3_spec(), _const_spec((1, E)), _const_spec((1, E)),
                      _const_spec((1, E)), _const_spec((1, E)),
                      _const_spec((E, HID)), _const_spec((1, HID)),
                      _const_spec((HID, E)), _const_spec((1, E))],
            out_specs=[_x3_spec(), _const_spec((1, E)), _const_spec((1, E))],
            out_shape=[x3_out, st_out, st_out],
        )(y3, s1, q1, gamma, beta, lp["W1"], lp["b1"].reshape(1, HID),
          lp["W2"], lp["b2"].reshape(1, E))
        stats = (s2, q2)
        prev_gb = (gamma, beta)

    xf, mf = pl.pallas_call(
        _out_body,
        grid=(STEPS,),
        in_specs=[_x3_spec(), _const_spec((1, E)), _const_spec((1, E)),
                  _const_spec((1, E)), _const_spec((1, E))],
        out_specs=[_x3_spec(), pl.BlockSpec((GB, 1, E), lambda i: (i, 0, 0))],
        out_shape=[x3_out, jax.ShapeDtypeStruct((B, 1, E), F32)],
    )(x3, stats[0], stats[1], prev_gb[0], prev_gb[1])

    return xf, mf.reshape(B, E)
